# initial kernel scaffold (unmeasured)
import jax
import jax.numpy as jnp
from jax import lax
from jax.experimental import pallas as pl
from jax.experimental.pallas import tpu as pltpu

N_DEV = 16


def kernel(x, w_mat):
    m_per, k = x.shape
    n_per = w_mat.shape[1]

    def body(x_ref, w_ref, out_ref, comm_ref, send_sems, recv_sems):
        me = lax.axis_index("i")
        right = (me + 1) % N_DEV
        left = (me - 1) % N_DEV

        barrier_sem = pltpu.get_barrier_semaphore()
        for nbr in (left, right):
            pl.semaphore_signal(
                barrier_sem, inc=1,
                device_id=(nbr,), device_id_type=pl.DeviceIdType.MESH,
            )
        pl.semaphore_wait(barrier_sem, 2)

        w_bf16 = w_ref[...].astype(jnp.bfloat16)

        comm_ref[me] = x_ref[...].astype(jnp.bfloat16)
        y = jnp.dot(comm_ref[me], w_bf16, preferred_element_type=jnp.float32)
        out_ref[pl.ds(me * m_per, m_per), :] = jnp.maximum(y, 0.0)

        for h in range(1, N_DEV):
            fwd = (me - h + 1) % N_DEV
            send = pltpu.make_async_remote_copy(
                src_ref=comm_ref.at[fwd],
                dst_ref=comm_ref.at[fwd],
                send_sem=send_sems.at[fwd],
                recv_sem=recv_sems.at[fwd],
                device_id=(right,),
                device_id_type=pl.DeviceIdType.MESH,
            )
            send.start()

            o = (me - h) % N_DEV
            recv = pltpu.make_async_remote_copy(
                src_ref=comm_ref.at[o],
                dst_ref=comm_ref.at[o],
                send_sem=send_sems.at[o],
                recv_sem=recv_sems.at[o],
                device_id=(left,),
                device_id_type=pl.DeviceIdType.MESH,
            )
            recv.wait_recv()
            send.wait_send()

            y = jnp.dot(comm_ref[o], w_bf16, preferred_element_type=jnp.float32)
            out_ref[pl.ds(o * m_per, m_per), :] = jnp.maximum(y, 0.0)

    return pl.pallas_call(
        body,
        out_shape=jax.ShapeDtypeStruct((N_DEV * m_per, n_per), jnp.float32),
        in_specs=[
            pl.BlockSpec(memory_space=pltpu.VMEM),
            pl.BlockSpec(memory_space=pltpu.VMEM),
        ],
        out_specs=pl.BlockSpec(memory_space=pltpu.VMEM),
        scratch_shapes=[
            pltpu.VMEM((N_DEV, m_per, k), jnp.bfloat16),
            pltpu.SemaphoreType.DMA((N_DEV,)),
            pltpu.SemaphoreType.DMA((N_DEV,)),
        ],
        compiler_params=pltpu.CompilerParams(collective_id=0),
    )(x, w_mat)


# baseline (device time: 387977 ns/iter reference)
import jax
import jax.numpy as jnp
from jax import lax
from jax.experimental import pallas as pl
from jax.experimental.pallas import tpu as pltpu

N_DEV = 16


def kernel(x, w_mat):
    m_per, k = x.shape
    n_per = w_mat.shape[1]

    def body(x_ref, w_ref, out_ref, comm_ref, send_sems, recv_sems):
        me = lax.axis_index("i")
        right = (me + 1) % N_DEV
        left = (me - 1) % N_DEV

        barrier_sem = pltpu.get_barrier_semaphore()
        for nbr in (left, right):
            pl.semaphore_signal(
                barrier_sem, inc=1,
                device_id=(nbr,), device_id_type=pl.DeviceIdType.MESH,
            )
        pl.semaphore_wait(barrier_sem, 2)

        w_bf16 = w_ref[...].astype(jnp.bfloat16)

        comm_ref[me] = x_ref[...].astype(jnp.bfloat16)
        y = jnp.dot(comm_ref[me], w_bf16, preferred_element_type=jnp.float32)
        out_ref[pl.ds(me * m_per, m_per), :] = jnp.maximum(y, 0.0)

        for h in range(1, N_DEV):
            fwd = (me - h + 1) % N_DEV
            send = pltpu.make_async_remote_copy(
                src_ref=comm_ref.at[fwd],
                dst_ref=comm_ref.at[fwd],
                send_sem=send_sems.at[fwd],
                recv_sem=recv_sems.at[fwd],
                device_id=(right,),
                device_id_type=pl.DeviceIdType.MESH,
            )
            send.start()

            o = (me - h) % N_DEV
            recv = pltpu.make_async_remote_copy(
                src_ref=comm_ref.at[o],
                dst_ref=comm_ref.at[o],
                send_sem=send_sems.at[o],
                recv_sem=recv_sems.at[o],
                device_id=(left,),
                device_id_type=pl.DeviceIdType.MESH,
            )
            recv.wait_recv()
            send.wait_send()

            y = jnp.dot(comm_ref[o], w_bf16, preferred_element_type=jnp.float32)
            out_ref[pl.ds(o * m_per, m_per), :] = jnp.maximum(y, 0.0)

    return pl.pallas_call(
        body,
        out_shape=jax.ShapeDtypeStruct((N_DEV * m_per, n_per), jnp.float32),
        in_specs=[
            pl.BlockSpec(memory_space=pltpu.VMEM),
            pl.BlockSpec(memory_space=pltpu.VMEM),
        ],
        out_specs=pl.BlockSpec(memory_space=pltpu.VMEM),
        scratch_shapes=[
            pltpu.VMEM((N_DEV, m_per, k), jnp.bfloat16),
            pltpu.SemaphoreType.DMA((N_DEV,)),
            pltpu.SemaphoreType.DMA((N_DEV,)),
        ],
        compiler_params=pltpu.CompilerParams(
            collective_id=0, vmem_limit_bytes=100 * 1024 * 1024
        ),
    )(x, w_mat)


# device time: 219520 ns/iter; 1.7674x vs baseline; 1.7674x over previous
import jax
import jax.numpy as jnp
from jax import lax
from jax.experimental import pallas as pl
from jax.experimental.pallas import tpu as pltpu

N_DEV = 16
R_HOPS = 8
L_HOPS = 7


def kernel(x, w_mat):
    m_per, k = x.shape
    n_per = w_mat.shape[1]

    def body(x_ref, w_ref, out_ref, comm_ref, ssr, rsr, ssl, rsl):
        me = lax.axis_index("i")
        right = (me + 1) % N_DEV
        left = (me - 1) % N_DEV

        barrier_sem = pltpu.get_barrier_semaphore()
        for nbr in (left, right):
            pl.semaphore_signal(
                barrier_sem, inc=1,
                device_id=(nbr,), device_id_type=pl.DeviceIdType.MESH,
            )
        pl.semaphore_wait(barrier_sem, 2)

        w_bf16 = w_ref[...].astype(jnp.bfloat16)
        comm_ref[me] = x_ref[...].astype(jnp.bfloat16)

        def gemm(o):
            y = jnp.dot(comm_ref[o], w_bf16, preferred_element_type=jnp.float32)
            out_ref[pl.ds(o * m_per, m_per), :] = jnp.maximum(y, 0.0)

        def make_rdma(o, sems_s, sems_r, dev):
            return pltpu.make_async_remote_copy(
                src_ref=comm_ref.at[o],
                dst_ref=comm_ref.at[o],
                send_sem=sems_s.at[o],
                recv_sem=sems_r.at[o],
                device_id=(dev,),
                device_id_type=pl.DeviceIdType.MESH,
            )

        for h in range(1, R_HOPS + 1):
            send_r = make_rdma((me - h + 1) % N_DEV, ssr, rsr, right)
            send_r.start()
            send_l = None
            if h <= L_HOPS:
                send_l = make_rdma((me + h - 1) % N_DEV, ssl, rsl, left)
                send_l.start()

            if h == 1:
                gemm(me)
            else:
                gemm((me - h + 1) % N_DEV)
                gemm((me + h - 1) % N_DEV)

            make_rdma((me - h) % N_DEV, ssr, rsr, left).wait_recv()
            if h <= L_HOPS:
                make_rdma((me + h) % N_DEV, ssl, rsl, right).wait_recv()
                send_l.wait_send()
            send_r.wait_send()

        gemm((me - R_HOPS) % N_DEV)
        gemm((me + L_HOPS) % N_DEV)

    return pl.pallas_call(
        body,
        out_shape=jax.ShapeDtypeStruct((N_DEV * m_per, n_per), jnp.float32),
        in_specs=[
            pl.BlockSpec(memory_space=pltpu.VMEM),
            pl.BlockSpec(memory_space=pltpu.VMEM),
        ],
        out_specs=pl.BlockSpec(memory_space=pltpu.VMEM),
        scratch_shapes=[
            pltpu.VMEM((N_DEV, m_per, k), jnp.bfloat16),
            pltpu.SemaphoreType.DMA((N_DEV,)),
            pltpu.SemaphoreType.DMA((N_DEV,)),
            pltpu.SemaphoreType.DMA((N_DEV,)),
            pltpu.SemaphoreType.DMA((N_DEV,)),
        ],
        compiler_params=pltpu.CompilerParams(
            collective_id=0, vmem_limit_bytes=100 * 1024 * 1024
        ),
    )(x, w_mat)


# device time: 207321 ns/iter; 1.8714x vs baseline; 1.0588x over previous
import jax
import jax.numpy as jnp
from jax import lax
from jax.experimental import pallas as pl
from jax.experimental.pallas import tpu as pltpu

N_DEV = 16
R_HOPS = 8
L_HOPS = 7

RING = (1, 5, 9, 13, 14, 10, 6, 2, 3, 7, 11, 15, 12, 8, 4, 0)
INV = tuple(RING.index(i) for i in range(N_DEV))


def _lookup(table, idx):
    v = jnp.int32(table[0])
    for j in range(1, len(table)):
        v = jnp.where(idx == j, jnp.int32(table[j]), v)
    return v


def kernel(x, w_mat):
    m_per, k = x.shape
    n_per = w_mat.shape[1]

    def body(x_ref, w_ref, out_ref, comm_ref, ssr, rsr, ssl, rsl):
        me = lax.axis_index("i")
        r = _lookup(INV, me)
        right = _lookup(RING, (r + 1) % N_DEV)
        left = _lookup(RING, (r - 1) % N_DEV)

        barrier_sem = pltpu.get_barrier_semaphore()
        for nbr in (left, right):
            pl.semaphore_signal(
                barrier_sem, inc=1,
                device_id=(nbr,), device_id_type=pl.DeviceIdType.MESH,
            )
        pl.semaphore_wait(barrier_sem, 2)

        w_bf16 = w_ref[...].astype(jnp.bfloat16)
        comm_ref[r] = x_ref[...].astype(jnp.bfloat16)

        def gemm(slot):
            off = _lookup(RING, slot) * m_per
            y = jnp.dot(
                comm_ref[slot], w_bf16, preferred_element_type=jnp.float32
            )
            out_ref[pl.ds(off, m_per), :] = jnp.maximum(y, 0.0)

        def make_rdma(slot, sems_s, sems_r, dev):
            return pltpu.make_async_remote_copy(
                src_ref=comm_ref.at[slot],
                dst_ref=comm_ref.at[slot],
                send_sem=sems_s.at[slot],
                recv_sem=sems_r.at[slot],
                device_id=(dev,),
                device_id_type=pl.DeviceIdType.MESH,
            )

        for h in range(1, R_HOPS + 1):
            send_r = make_rdma((r - h + 1) % N_DEV, ssr, rsr, right)
            send_r.start()
            send_l = None
            if h <= L_HOPS:
                send_l = make_rdma((r + h - 1) % N_DEV, ssl, rsl, left)
                send_l.start()

            if h == 1:
                gemm(r)
            else:
                gemm((r - h + 1) % N_DEV)
                gemm((r + h - 1) % N_DEV)

            make_rdma((r - h) % N_DEV, ssr, rsr, left).wait_recv()
            if h <= L_HOPS:
                make_rdma((r + h) % N_DEV, ssl, rsl, right).wait_recv()
                send_l.wait_send()
            send_r.wait_send()

        gemm((r - R_HOPS) % N_DEV)
        gemm((r + L_HOPS) % N_DEV)

    return pl.pallas_call(
        body,
        out_shape=jax.ShapeDtypeStruct((N_DEV * m_per, n_per), jnp.float32),
        in_specs=[
            pl.BlockSpec(memory_space=pltpu.VMEM),
            pl.BlockSpec(memory_space=pltpu.VMEM),
        ],
        out_specs=pl.BlockSpec(memory_space=pltpu.VMEM),
        scratch_shapes=[
            pltpu.VMEM((N_DEV, m_per, k), jnp.bfloat16),
            pltpu.SemaphoreType.DMA((N_DEV,)),
            pltpu.SemaphoreType.DMA((N_DEV,)),
            pltpu.SemaphoreType.DMA((N_DEV,)),
            pltpu.SemaphoreType.DMA((N_DEV,)),
        ],
        compiler_params=pltpu.CompilerParams(
            collective_id=0, vmem_limit_bytes=100 * 1024 * 1024
        ),
    )(x, w_mat)


# device time: 196098 ns/iter; 1.9785x vs baseline; 1.0572x over previous
import jax
import jax.numpy as jnp
from jax import lax
from jax.experimental import pallas as pl
from jax.experimental.pallas import tpu as pltpu

N_DEV = 16
N_HOPS = 8

RING = (1, 5, 9, 13, 14, 10, 6, 2, 3, 7, 11, 15, 12, 8, 4, 0)
INV = tuple(RING.index(i) for i in range(N_DEV))


def _lookup(table, idx):
    v = jnp.int32(table[0])
    for j in range(1, len(table)):
        v = jnp.where(idx == j, jnp.int32(table[j]), v)
    return v


def kernel(x, w_mat):
    m_per, k = x.shape
    n_per = w_mat.shape[1]
    m_half = m_per // 2

    def body(x_ref, w_ref, out_ref, comm_ref, ssr, rsr, ssl, rsl):
        me = lax.axis_index("i")
        r = _lookup(INV, me)
        right = _lookup(RING, (r + 1) % N_DEV)
        left = _lookup(RING, (r - 1) % N_DEV)

        barrier_sem = pltpu.get_barrier_semaphore()
        for nbr in (left, right):
            pl.semaphore_signal(
                barrier_sem, inc=1,
                device_id=(nbr,), device_id_type=pl.DeviceIdType.MESH,
            )
        pl.semaphore_wait(barrier_sem, 2)

        w_bf16 = w_ref[...].astype(jnp.bfloat16)
        comm_ref[pl.ds(2 * r, 2)] = x_ref[...].astype(jnp.bfloat16).reshape(
            2, m_half, k
        )

        def gemm_half(hs):
            off = _lookup(RING, hs // 2) * m_per + (hs % 2) * m_half
            y = jnp.dot(
                comm_ref[hs], w_bf16, preferred_element_type=jnp.float32
            )
            out_ref[pl.ds(off, m_half), :] = jnp.maximum(y, 0.0)

        def gemm_slot(s):
            gemm_half(2 * s)
            gemm_half(2 * s + 1)

        def make_rdma(hs, sems_s, sems_r, dev):
            return pltpu.make_async_remote_copy(
                src_ref=comm_ref.at[hs],
                dst_ref=comm_ref.at[hs],
                send_sem=sems_s.at[hs],
                recv_sem=sems_r.at[hs],
                device_id=(dev,),
                device_id_type=pl.DeviceIdType.MESH,
            )

        for h in range(1, N_HOPS + 1):
            sends = []
            fwd_r = (r - h + 1) % N_DEV
            fwd_l = (r + h - 1) % N_DEV
            sends.append(make_rdma(2 * fwd_r, ssr, rsr, right))
            sends.append(make_rdma(2 * fwd_l + 1, ssl, rsl, left))
            if h < N_HOPS:
                sends.append(make_rdma(2 * fwd_r + 1, ssr, rsr, right))
                sends.append(make_rdma(2 * fwd_l, ssl, rsl, left))
            for s_ in sends:
                s_.start()

            if h == 1:
                gemm_slot(r)
            else:
                gemm_slot((r - h + 1) % N_DEV)
                gemm_slot((r + h - 1) % N_DEV)

            rcv_r = (r - h) % N_DEV
            rcv_l = (r + h) % N_DEV
            make_rdma(2 * rcv_r, ssr, rsr, left).wait_recv()
            make_rdma(2 * rcv_l + 1, ssl, rsl, right).wait_recv()
            if h < N_HOPS:
                make_rdma(2 * rcv_r + 1, ssr, rsr, left).wait_recv()
                make_rdma(2 * rcv_l, ssl, rsl, right).wait_recv()
            for s_ in sends:
                s_.wait_send()

        gemm_slot((r - N_HOPS) % N_DEV)

    return pl.pallas_call(
        body,
        out_shape=jax.ShapeDtypeStruct((N_DEV * m_per, n_per), jnp.float32),
        in_specs=[
            pl.BlockSpec(memory_space=pltpu.VMEM),
            pl.BlockSpec(memory_space=pltpu.VMEM),
        ],
        out_specs=pl.BlockSpec(memory_space=pltpu.VMEM),
        scratch_shapes=[
            pltpu.VMEM((2 * N_DEV, m_half, k), jnp.bfloat16),
            pltpu.SemaphoreType.DMA((2 * N_DEV,)),
            pltpu.SemaphoreType.DMA((2 * N_DEV,)),
            pltpu.SemaphoreType.DMA((2 * N_DEV,)),
            pltpu.SemaphoreType.DMA((2 * N_DEV,)),
        ],
        compiler_params=pltpu.CompilerParams(
            collective_id=0, vmem_limit_bytes=100 * 1024 * 1024
        ),
    )(x, w_mat)


# device time: 182993 ns/iter; 2.1202x vs baseline; 1.0716x over previous
import jax
import jax.numpy as jnp
from jax import lax
from jax.experimental import pallas as pl
from jax.experimental.pallas import tpu as pltpu

N_DEV = 16
N_HOPS = 8

RING = (1, 5, 9, 13, 14, 10, 6, 2, 3, 7, 11, 15, 12, 8, 4, 0)
INV = tuple(RING.index(i) for i in range(N_DEV))


def _lookup(table, idx):
    v = jnp.int32(table[0])
    for j in range(1, len(table)):
        v = jnp.where(idx == j, jnp.int32(table[j]), v)
    return v


def kernel(x, w_mat):
    m_per, k = x.shape
    n_per = w_mat.shape[1]
    m_half = m_per // 2

    def body(x_ref, w_ref, out_ref, comm_ref, ssr, rsr, ssl, rsl):
        me = lax.axis_index("i")
        r = _lookup(INV, me)
        right = _lookup(RING, (r + 1) % N_DEV)
        left = _lookup(RING, (r - 1) % N_DEV)

        barrier_sem = pltpu.get_barrier_semaphore()
        for nbr in (left, right):
            pl.semaphore_signal(
                barrier_sem, inc=1,
                device_id=(nbr,), device_id_type=pl.DeviceIdType.MESH,
            )
        pl.semaphore_wait(barrier_sem, 2)

        w_bf16 = w_ref[...].astype(jnp.bfloat16)
        comm_ref[pl.ds(2 * r, 2)] = x_ref[...].astype(jnp.bfloat16).reshape(
            2, m_half, k
        )

        def gemm_half(hs):
            off = _lookup(RING, hs // 2) * m_per + (hs % 2) * m_half
            y = jnp.dot(
                comm_ref[hs], w_bf16, preferred_element_type=jnp.float32
            )
            out_ref[pl.ds(off, m_half), :] = jnp.maximum(y, 0.0)

        def make_rdma(hs, sems_s, sems_r, dev):
            return pltpu.make_async_remote_copy(
                src_ref=comm_ref.at[hs],
                dst_ref=comm_ref.at[hs],
                send_sem=sems_s.at[hs],
                recv_sem=sems_r.at[hs],
                device_id=(dev,),
                device_id_type=pl.DeviceIdType.MESH,
            )

        pending_sends = []

        def send(hs, sems_s, sems_r, dev):
            rdma = make_rdma(hs, sems_s, sems_r, dev)
            rdma.start()
            pending_sends.append(rdma)

        send(2 * r, ssr, rsr, right)
        send(2 * r + 1, ssl, rsl, left)
        send(2 * r + 1, ssr, rsr, right)
        send(2 * r, ssl, rsl, left)
        gemm_half(2 * r)
        gemm_half(2 * r + 1)

        for h in range(1, N_HOPS + 1):
            rs = (r - h) % N_DEV
            ls = (r + h) % N_DEV

            make_rdma(2 * rs, ssr, rsr, left).wait_recv()
            if h < N_HOPS:
                send(2 * rs, ssr, rsr, right)
            gemm_half(2 * rs)

            make_rdma(2 * ls + 1, ssl, rsl, right).wait_recv()
            if h < N_HOPS:
                send(2 * ls + 1, ssl, rsl, left)
            gemm_half(2 * ls + 1)

            if h < N_HOPS:
                make_rdma(2 * rs + 1, ssr, rsr, left).wait_recv()
                if h < N_HOPS - 1:
                    send(2 * rs + 1, ssr, rsr, right)
                gemm_half(2 * rs + 1)

                make_rdma(2 * ls, ssl, rsl, right).wait_recv()
                if h < N_HOPS - 1:
                    send(2 * ls, ssl, rsl, left)
                gemm_half(2 * ls)

        for rdma in pending_sends:
            rdma.wait_send()

    return pl.pallas_call(
        body,
        out_shape=jax.ShapeDtypeStruct((N_DEV * m_per, n_per), jnp.float32),
        in_specs=[
            pl.BlockSpec(memory_space=pltpu.VMEM),
            pl.BlockSpec(memory_space=pltpu.VMEM),
        ],
        out_specs=pl.BlockSpec(memory_space=pltpu.VMEM),
        scratch_shapes=[
            pltpu.VMEM((2 * N_DEV, m_half, k), jnp.bfloat16),
            pltpu.SemaphoreType.DMA((2 * N_DEV,)),
            pltpu.SemaphoreType.DMA((2 * N_DEV,)),
            pltpu.SemaphoreType.DMA((2 * N_DEV,)),
            pltpu.SemaphoreType.DMA((2 * N_DEV,)),
        ],
        compiler_params=pltpu.CompilerParams(
            collective_id=0, vmem_limit_bytes=100 * 1024 * 1024
        ),
    )(x, w_mat)
